# baseline (speedup 1.0000x reference)
import functools

import jax
import jax.numpy as jnp
from jax import lax
from jax.experimental import pallas as pl
from jax.experimental.pallas import tpu as pltpu
from jax.experimental.pallas import tpu_sc as plsc

B = 8
N = 8192
G = 512
K = 32
L = 16
NQ = 4
NPQ = N // NQ
RPQ = G // NQ
NCHUNK = N // L
NB = 1024
INT_MAX = 0x7FFFFFFF


def _splat(x, dtype=None):
    v = jnp.broadcast_to(x, (L,))
    return v if dtype is None else v.astype(dtype)


def _sc_group_kernel(xyz_hbm, neigh_hbm, cent_hbm,
                     x_ref, y_ref, z_ref, d_ref,
                     cxs, cys, czs,
                     out_stage, cent_stage,
                     rf_ref, rinf_ref, cand_d, cand_i, shf):
    c = lax.axis_index("c")
    s = lax.axis_index("s")
    w = c * 16 + s
    b = w // NQ
    p = w % NQ
    iota = lax.iota(jnp.int32, L)
    lane0 = iota == 0
    base_pt = p * NPQ

    pltpu.sync_copy(xyz_hbm.at[pl.ds((b * 3 + 0) * N, N)], x_ref)
    pltpu.sync_copy(xyz_hbm.at[pl.ds((b * 3 + 1) * N, N)], y_ref)
    pltpu.sync_copy(xyz_hbm.at[pl.ds((b * 3 + 2) * N, N)], z_ref)

    def init_d(j, carry):
        d_ref[pl.ds(j * L, L)] = jnp.full((L,), 1e10, jnp.float32)
        return carry

    lax.fori_loop(0, NPQ // L, init_d, 0)

    def fps_iter(i, carry):
        cxv, cyv, czv, _ = carry
        ivec = _splat(i).astype(jnp.int32)
        plsc.store_scatter(cxs, [ivec], cxv, mask=lane0)
        plsc.store_scatter(cys, [ivec], cyv, mask=lane0)
        plsc.store_scatter(czs, [ivec], czv, mask=lane0)

        @plsc.parallel_loop(0, NPQ // L, 4, unroll=2,
                            carry=(jnp.full((L,), -1.0, jnp.float32),
                                   jnp.zeros((L,), jnp.int32)))
        def chunk(j, cc):
            bestv, besti = cc
            for u in range(4):
                jj = j + u
                offs = base_pt + jj * L
                xv = x_ref[pl.ds(offs, L)]
                yv = y_ref[pl.ds(offs, L)]
                zv = z_ref[pl.ds(offs, L)]
                dx = xv - cxv
                dy = yv - cyv
                dz = zv - czv
                dist = dx * dx + dy * dy + dz * dz
                dold = d_ref[pl.ds(jj * L, L)]
                dnew = jnp.minimum(dold, dist)
                d_ref[pl.ds(jj * L, L)] = dnew
                idxv = offs + iota
                upd = dnew > bestv
                bestv = jnp.where(upd, dnew, bestv)
                besti = jnp.where(upd, idxv, besti)
            return (bestv, besti)

        bestv, besti = chunk
        mx = jnp.max(bestv)
        lidx = jnp.min(jnp.where(bestv == mx, besti, INT_MAX))

        ints = jnp.where(iota < 12, _splat(lidx), _splat(i)).astype(jnp.int32)
        rec = jnp.where(iota < 8, _splat(mx), plsc.bitcast(ints, jnp.float32))
        rf_ref[...] = rec
        par = (i & 1) * 32
        pltpu.sync_copy(rf_ref, shf.at[pl.ds((par + w) * L, L)])
        plsc.subcore_barrier()
        g0 = (w // NQ) * NQ
        rows = (iota & (NQ - 1)) * L

        def rd_body(tmin):
            pltpu.sync_copy(shf.at[pl.ds((par + g0) * L, NQ * L)], rinf_ref)
            tags = plsc.bitcast(plsc.load_gather(rinf_ref, [rows + 12]),
                                jnp.int32)
            return jnp.min(tags)

        lax.while_loop(lambda tmin: tmin < i, rd_body, jnp.int32(-1))
        mxs = plsc.load_gather(rinf_ref, [rows])
        idxs = plsc.bitcast(plsc.load_gather(rinf_ref, [rows + 8]), jnp.int32)
        gmx = jnp.max(mxs)
        widx = jnp.min(jnp.where(mxs == gmx, idxs, INT_MAX))
        wv = _splat(widx).astype(jnp.int32)
        ncx = plsc.load_gather(x_ref, [wv])
        ncy = plsc.load_gather(y_ref, [wv])
        ncz = plsc.load_gather(z_ref, [wv])
        return (ncx, ncy, ncz, _splat(gmx))

    rf_ref[...] = plsc.bitcast(jnp.full((L,), -1, jnp.int32), jnp.float32)
    pltpu.sync_copy(rf_ref, shf.at[pl.ds(w * L, L)])
    pltpu.sync_copy(rf_ref, shf.at[pl.ds((32 + w) * L, L)])
    plsc.subcore_barrier()

    cx0 = _splat(jnp.sum(jnp.where(lane0, x_ref[pl.ds(0, L)], 0.0)))
    cy0 = _splat(jnp.sum(jnp.where(lane0, y_ref[pl.ds(0, L)], 0.0)))
    cz0 = _splat(jnp.sum(jnp.where(lane0, z_ref[pl.ds(0, L)], 0.0)))
    zf = jnp.zeros((L,), jnp.float32)
    _, _, _, gmx_v = lax.fori_loop(0, G, fps_iter, (cx0, cy0, cz0, zf))

    row0 = p * RPQ

    def row_body(r, thr_in):
        rvec = _splat(row0 + r).astype(jnp.int32)
        cxv = plsc.load_gather(cxs, [rvec])
        cyv = plsc.load_gather(cys, [rvec])
        czv = plsc.load_gather(czs, [rvec])

        @plsc.parallel_loop(0, NCHUNK, 4, unroll=2,
                            carry=jnp.zeros((L,), jnp.int32))
        def pass1(cn, off_v):
            for u in range(4):
                cc2 = cn + u
                sl = pl.ds(cc2 * L, L)
                dx = x_ref[sl] - cxv
                dy = y_ref[sl] - cyv
                dz = z_ref[sl] - czv
                d = dx * dx + dy * dy + dz * dz
                m = d <= thr_in
                pos = off_v * L + iota
                plsc.store_scatter(cand_d, [pos], d, mask=m)
                plsc.store_scatter(cand_i, [pos], cc2 * L + iota, mask=m)
                off_v = off_v + m.astype(jnp.int32)
            return off_v

        offs_v = pass1
        c0 = jnp.sum(offs_v)

        def grow_body(cc):
            thr_c, _ = cc
            thr_n = thr_c * 4.0 + 1e-30

            def recount(cn, cv):
                for u in range(4):
                    sl = pl.ds((cn * 4 + u) * L, L)
                    dx = x_ref[sl] - cxv
                    dy = y_ref[sl] - cyv
                    dz = z_ref[sl] - czv
                    d = dx * dx + dy * dy + dz * dz
                    cv = cv + (d <= thr_n).astype(jnp.int32)
                return cv

            cv = lax.fori_loop(0, NCHUNK // 4, recount,
                               jnp.zeros((L,), jnp.int32))
            return (thr_n, jnp.sum(cv))

        def reselect(_):
            thr_sel, _c = lax.while_loop(lambda cc: cc[1] < K, grow_body,
                                         (thr_in, c0))

            def reappend(cn, off_v):
                for u in range(4):
                    cc2 = cn * 4 + u
                    sl = pl.ds(cc2 * L, L)
                    dx = x_ref[sl] - cxv
                    dy = y_ref[sl] - cyv
                    dz = z_ref[sl] - czv
                    d = dx * dx + dy * dy + dz * dz
                    m = d <= thr_sel
                    pos = off_v * L + iota
                    plsc.store_scatter(cand_d, [pos], d, mask=m)
                    plsc.store_scatter(cand_i, [pos], cc2 * L + iota, mask=m)
                    off_v = off_v + m.astype(jnp.int32)
                return off_v

            return lax.fori_loop(0, NCHUNK // 4, reappend,
                                 jnp.zeros((L,), jnp.int32))

        offs_v = lax.cond(c0 < K, reselect, lambda _: offs_v, 0)
        maxlen = jnp.max(offs_v)

        inf = jnp.full((L,), jnp.inf, jnp.float32)
        inf = jnp.full((L,), jnp.inf, jnp.float32)

        def pass3(ci, cc):
            t0k, t0v, t1k, t1v = cc
            d = cand_d[pl.ds(ci * L, L)]
            idxv = cand_i[pl.ds(ci * L, L)]
            d = jnp.where(ci < offs_v, d, jnp.inf)
            sk, sv = plsc.sort_key_val(d, idxv)
            rk = lax.rev(sk, (0,))
            rv = lax.rev(sv, (0,))
            m = t1k <= rk
            lok = jnp.where(m, t1k, rk)
            lov = jnp.where(m, t1v, rv)
            s1k, s1v = plsc.sort_key_val(lok, lov)
            r2k = lax.rev(s1k, (0,))
            r2v = lax.rev(s1v, (0,))
            m2 = t0k <= r2k
            l2k = jnp.where(m2, t0k, r2k)
            l2v = jnp.where(m2, t0v, r2v)
            h2k = jnp.where(m2, r2k, t0k)
            h2v = jnp.where(m2, r2v, t0v)
            nt0k, nt0v = plsc.sort_key_val(l2k, l2v)
            nt1k, nt1v = plsc.sort_key_val(h2k, h2v)
            return (nt0k, nt0v, nt1k, nt1v)

        t0k, t0v, t1k, t1v = lax.fori_loop(
            0, maxlen, pass3,
            (inf, jnp.zeros((L,), jnp.int32), inf,
             jnp.zeros((L,), jnp.int32)))
        thr_out = _splat(jnp.max(t1k)) * 2.0

        for half, tv in ((0, t0v), (1, t1v)):
            gx = plsc.load_gather(x_ref, [tv]) - cxv
            gy = plsc.load_gather(y_ref, [tv]) - cyv
            gz = plsc.load_gather(z_ref, [tv]) - czv
            kpos = (r * K + half * L + iota) * 3
            plsc.store_scatter(out_stage, [kpos], gx)
            plsc.store_scatter(out_stage, [kpos + 1], gy)
            plsc.store_scatter(out_stage, [kpos + 2], gz)
        return thr_out

    lax.fori_loop(0, RPQ, row_body, gmx_v * 4.0)
    pltpu.sync_copy(out_stage,
                    neigh_hbm.at[pl.ds((b * G + row0) * K * 3, RPQ * K * 3)])

    @pl.when(p == 0)
    def _():
        def cent_chunk(i, carry):
            idxv = (i * L + iota) * 3
            for coord, ref in ((0, cxs), (1, cys), (2, czs)):
                v = ref[pl.ds(i * L, L)]
                plsc.store_scatter(cent_stage, [idxv + coord], v)
            return carry

        lax.fori_loop(0, G // L, cent_chunk, 0)
        pltpu.sync_copy(cent_stage, cent_hbm.at[pl.ds(b * G * 3, G * 3)])


@jax.jit
def kernel(xyz):
    xyz_flat = jnp.transpose(xyz, (0, 2, 1)).reshape(-1)
    mesh = plsc.VectorSubcoreMesh(core_axis_name="c", subcore_axis_name="s")
    fn = functools.partial(
        pl.kernel,
        out_type=(jax.ShapeDtypeStruct((B * G * K * 3,), jnp.float32),
                  jax.ShapeDtypeStruct((B * G * 3,), jnp.float32)),
        mesh=mesh,
        compiler_params=pltpu.CompilerParams(needs_layout_passes=False),
        scratch_types=[
            pltpu.VMEM((N,), jnp.float32),
            pltpu.VMEM((N,), jnp.float32),
            pltpu.VMEM((N,), jnp.float32),
            pltpu.VMEM((NPQ,), jnp.float32),
            pltpu.VMEM((G,), jnp.float32),
            pltpu.VMEM((G,), jnp.float32),
            pltpu.VMEM((G,), jnp.float32),
            pltpu.VMEM((RPQ * K * 3,), jnp.float32),
            pltpu.VMEM((G * 3,), jnp.float32),
            pltpu.VMEM((L,), jnp.float32),
            pltpu.VMEM((NQ * L,), jnp.float32),
            pltpu.VMEM((N + L,), jnp.float32),
            pltpu.VMEM((N + L,), jnp.int32),
            pltpu.VMEM_SHARED((2 * 32 * L,), jnp.float32),
        ],
    )(_sc_group_kernel)
    neigh, cent = fn(xyz_flat)
    return neigh.reshape(B, G, K, 3), cent.reshape(B, G, 3)


# speedup vs baseline: 1.8704x; 1.8704x over previous
import functools

import jax
import jax.numpy as jnp
from jax import lax
from jax.experimental import pallas as pl
from jax.experimental.pallas import tpu as pltpu
from jax.experimental.pallas import tpu_sc as plsc

B = 8
N = 8192
G = 512
K = 32
L = 16
NQ = 4
NPQ = N // NQ
RPQ = G // NQ
NCHUNK = N // L
NB = 1024
INT_MAX = 0x7FFFFFFF


def _splat(x, dtype=None):
    v = jnp.broadcast_to(x, (L,))
    return v if dtype is None else v.astype(dtype)


def _sc_group_kernel(xyz_hbm, neigh_hbm, cent_hbm,
                     x_ref, y_ref, z_ref, d_ref,
                     cxs, cys, czs,
                     out_stage, cent_stage,
                     rf_ref, rinf_ref, drow_ref, cand_d, cand_i, shf):
    c = lax.axis_index("c")
    s = lax.axis_index("s")
    w = c * 16 + s
    b = w // NQ
    p = w % NQ
    iota = lax.iota(jnp.int32, L)
    lane0 = iota == 0
    base_pt = p * NPQ

    pltpu.sync_copy(xyz_hbm.at[pl.ds((b * 3 + 0) * N, N)], x_ref)
    pltpu.sync_copy(xyz_hbm.at[pl.ds((b * 3 + 1) * N, N)], y_ref)
    pltpu.sync_copy(xyz_hbm.at[pl.ds((b * 3 + 2) * N, N)], z_ref)

    def init_d(j, carry):
        d_ref[pl.ds(j * L, L)] = jnp.full((L,), 1e10, jnp.float32)
        return carry

    lax.fori_loop(0, NPQ // L, init_d, 0)

    def fps_iter(i, carry):
        cxv, cyv, czv, _ = carry
        ivec = _splat(i).astype(jnp.int32)
        plsc.store_scatter(cxs, [ivec], cxv, mask=lane0)
        plsc.store_scatter(cys, [ivec], cyv, mask=lane0)
        plsc.store_scatter(czs, [ivec], czv, mask=lane0)

        @plsc.parallel_loop(0, NPQ // L, 4, unroll=2,
                            carry=(jnp.full((L,), -1.0, jnp.float32),
                                   jnp.zeros((L,), jnp.int32)))
        def chunk(j, cc):
            bestv, besti = cc
            for u in range(4):
                jj = j + u
                offs = base_pt + jj * L
                xv = x_ref[pl.ds(offs, L)]
                yv = y_ref[pl.ds(offs, L)]
                zv = z_ref[pl.ds(offs, L)]
                dx = xv - cxv
                dy = yv - cyv
                dz = zv - czv
                dist = dx * dx + dy * dy + dz * dz
                dold = d_ref[pl.ds(jj * L, L)]
                dnew = jnp.minimum(dold, dist)
                d_ref[pl.ds(jj * L, L)] = dnew
                idxv = offs + iota
                upd = dnew > bestv
                bestv = jnp.where(upd, dnew, bestv)
                besti = jnp.where(upd, idxv, besti)
            return (bestv, besti)

        bestv, besti = chunk
        mx = jnp.max(bestv)
        lidx = jnp.min(jnp.where(bestv == mx, besti, INT_MAX))

        ints = jnp.where(iota < 12, _splat(lidx), _splat(i)).astype(jnp.int32)
        rec = jnp.where(iota < 8, _splat(mx), plsc.bitcast(ints, jnp.float32))
        rf_ref[...] = rec
        par = (i & 1) * 32
        pltpu.sync_copy(rf_ref, shf.at[pl.ds((par + w) * L, L)])
        plsc.subcore_barrier()
        g0 = (w // NQ) * NQ
        rows = (iota & (NQ - 1)) * L

        def rd_body(tmin):
            pltpu.sync_copy(shf.at[pl.ds((par + g0) * L, NQ * L)], rinf_ref)
            tags = plsc.bitcast(plsc.load_gather(rinf_ref, [rows + 12]),
                                jnp.int32)
            return jnp.min(tags)

        lax.while_loop(lambda tmin: tmin < i, rd_body, jnp.int32(-1))
        mxs = plsc.load_gather(rinf_ref, [rows])
        idxs = plsc.bitcast(plsc.load_gather(rinf_ref, [rows + 8]), jnp.int32)
        gmx = jnp.max(mxs)
        widx = jnp.min(jnp.where(mxs == gmx, idxs, INT_MAX))
        wv = _splat(widx).astype(jnp.int32)
        ncx = plsc.load_gather(x_ref, [wv])
        ncy = plsc.load_gather(y_ref, [wv])
        ncz = plsc.load_gather(z_ref, [wv])
        return (ncx, ncy, ncz, _splat(gmx))

    rf_ref[...] = plsc.bitcast(jnp.full((L,), -1, jnp.int32), jnp.float32)
    pltpu.sync_copy(rf_ref, shf.at[pl.ds(w * L, L)])
    pltpu.sync_copy(rf_ref, shf.at[pl.ds((32 + w) * L, L)])
    plsc.subcore_barrier()

    cx0 = _splat(jnp.sum(jnp.where(lane0, x_ref[pl.ds(0, L)], 0.0)))
    cy0 = _splat(jnp.sum(jnp.where(lane0, y_ref[pl.ds(0, L)], 0.0)))
    cz0 = _splat(jnp.sum(jnp.where(lane0, z_ref[pl.ds(0, L)], 0.0)))
    zf = jnp.zeros((L,), jnp.float32)
    _, _, _, gmx_v = lax.fori_loop(0, G, fps_iter, (cx0, cy0, cz0, zf))

    row0 = p * RPQ

    def row_body(r, thr_in):
        rvec = _splat(row0 + r).astype(jnp.int32)
        cxv = plsc.load_gather(cxs, [rvec])
        cyv = plsc.load_gather(cys, [rvec])
        czv = plsc.load_gather(czs, [rvec])

        @plsc.parallel_loop(0, NCHUNK, 4, unroll=2,
                            carry=jnp.zeros((L,), jnp.int32))
        def pass1(cn, cnt_v):
            for u in range(4):
                sl = pl.ds((cn + u) * L, L)
                dx = x_ref[sl] - cxv
                dy = y_ref[sl] - cyv
                dz = z_ref[sl] - czv
                d = dx * dx + dy * dy + dz * dz
                drow_ref[sl] = d
                cnt_v = cnt_v + (d <= thr_in).astype(jnp.int32)
            return cnt_v

        c0 = jnp.sum(pass1)

        def grow_body(cc):
            thr_c, _ = cc
            thr_n = thr_c * 4.0 + 1e-30

            def recount(cn, cv):
                for u in range(4):
                    sl = pl.ds((cn * 4 + u) * L, L)
                    cv = cv + (drow_ref[sl] <= thr_n).astype(jnp.int32)
                return cv

            cv = lax.fori_loop(0, NCHUNK // 4, recount,
                               jnp.zeros((L,), jnp.int32))
            return (thr_n, jnp.sum(cv))

        thr_sel, _ = lax.while_loop(lambda cc: cc[1] < K, grow_body,
                                    (thr_in, c0))

        NQC = NCHUNK // 4

        @plsc.parallel_loop(0, NQC, 1, unroll=2,
                            carry=(jnp.int32(0), jnp.int32(0),
                                   jnp.int32(0), jnp.int32(0)))
        def pass2(cn, offs):
            new_offs = []
            for q in range(4):
                off = offs[q]
                cc2 = q * NQC + cn
                sl = pl.ds(cc2 * L, L)
                d = drow_ref[sl]
                m = d <= thr_sel
                idxv = cc2 * L + iota
                base = q * (N // 4)
                plsc.store_compressed(cand_d.at[pl.ds(base + off, L)], d,
                                      mask=m)
                plsc.store_compressed(cand_i.at[pl.ds(base + off, L)], idxv,
                                      mask=m)
                new_offs.append(off + plsc.all_reduce_population_count(m)[0])
            return tuple(new_offs)

        qsizes = pass2

        inf = jnp.full((L,), jnp.inf, jnp.float32)
        acc = (inf, jnp.zeros((L,), jnp.int32), inf,
               jnp.zeros((L,), jnp.int32))
        for q in range(4):
            qbase = q * (N // 4)
            qn = qsizes[q]
            qv = _splat(qn)

            def pass3(ci, cc, qbase=qbase, qv=qv):
                t0k, t0v, t1k, t1v = cc
                d = cand_d[pl.ds(qbase + ci * L, L)]
                idxv = cand_i[pl.ds(qbase + ci * L, L)]
                d = jnp.where(ci * L + iota < qv, d, jnp.inf)
                sk, sv = plsc.sort_key_val(d, idxv)
                rk = lax.rev(sk, (0,))
                rv = lax.rev(sv, (0,))
                m = t1k <= rk
                lok = jnp.where(m, t1k, rk)
                lov = jnp.where(m, t1v, rv)
                s1k, s1v = plsc.sort_key_val(lok, lov)
                r2k = lax.rev(s1k, (0,))
                r2v = lax.rev(s1v, (0,))
                m2 = t0k <= r2k
                l2k = jnp.where(m2, t0k, r2k)
                l2v = jnp.where(m2, t0v, r2v)
                h2k = jnp.where(m2, r2k, t0k)
                h2v = jnp.where(m2, r2v, t0v)
                nt0k, nt0v = plsc.sort_key_val(l2k, l2v)
                nt1k, nt1v = plsc.sort_key_val(h2k, h2v)
                return (nt0k, nt0v, nt1k, nt1v)

            acc = lax.fori_loop(0, (qn + L - 1) // L, pass3, acc)
        t0k, t0v, t1k, t1v = acc
        thr_out = _splat(jnp.max(t1k)) * 2.0

        for half, tv in ((0, t0v), (1, t1v)):
            gx = plsc.load_gather(x_ref, [tv]) - cxv
            gy = plsc.load_gather(y_ref, [tv]) - cyv
            gz = plsc.load_gather(z_ref, [tv]) - czv
            kpos = (r * K + half * L + iota) * 3
            plsc.store_scatter(out_stage, [kpos], gx)
            plsc.store_scatter(out_stage, [kpos + 1], gy)
            plsc.store_scatter(out_stage, [kpos + 2], gz)
        return thr_out

    lax.fori_loop(0, RPQ, row_body, gmx_v * 4.0)
    pltpu.sync_copy(out_stage,
                    neigh_hbm.at[pl.ds((b * G + row0) * K * 3, RPQ * K * 3)])

    @pl.when(p == 0)
    def _():
        def cent_chunk(i, carry):
            idxv = (i * L + iota) * 3
            for coord, ref in ((0, cxs), (1, cys), (2, czs)):
                v = ref[pl.ds(i * L, L)]
                plsc.store_scatter(cent_stage, [idxv + coord], v)
            return carry

        lax.fori_loop(0, G // L, cent_chunk, 0)
        pltpu.sync_copy(cent_stage, cent_hbm.at[pl.ds(b * G * 3, G * 3)])


@jax.jit
def kernel(xyz):
    xyz_flat = jnp.transpose(xyz, (0, 2, 1)).reshape(-1)
    mesh = plsc.VectorSubcoreMesh(core_axis_name="c", subcore_axis_name="s")
    fn = functools.partial(
        pl.kernel,
        out_type=(jax.ShapeDtypeStruct((B * G * K * 3,), jnp.float32),
                  jax.ShapeDtypeStruct((B * G * 3,), jnp.float32)),
        mesh=mesh,
        compiler_params=pltpu.CompilerParams(needs_layout_passes=False),
        scratch_types=[
            pltpu.VMEM((N,), jnp.float32),
            pltpu.VMEM((N,), jnp.float32),
            pltpu.VMEM((N,), jnp.float32),
            pltpu.VMEM((NPQ,), jnp.float32),
            pltpu.VMEM((G,), jnp.float32),
            pltpu.VMEM((G,), jnp.float32),
            pltpu.VMEM((G,), jnp.float32),
            pltpu.VMEM((RPQ * K * 3,), jnp.float32),
            pltpu.VMEM((G * 3,), jnp.float32),
            pltpu.VMEM((L,), jnp.float32),
            pltpu.VMEM((NQ * L,), jnp.float32),
            pltpu.VMEM((N,), jnp.float32),
            pltpu.VMEM((N + L,), jnp.float32),
            pltpu.VMEM((N + L,), jnp.int32),
            pltpu.VMEM_SHARED((2 * 32 * L,), jnp.float32),
        ],
    )(_sc_group_kernel)
    neigh, cent = fn(xyz_flat)
    return neigh.reshape(B, G, K, 3), cent.reshape(B, G, 3)
